# baseline (speedup 1.0000x reference)
import functools
import random as _pyrandom

import numpy as np
import jax
import jax.numpy as jnp
from jax import lax
from jax.experimental import pallas as pl
from jax.experimental.pallas import tpu as pltpu
from jax.experimental.pallas import tpu_sc as plsc

_RF = 8
_N_POSITIVE = 2
_K = 1
_MARGIN = 12.0
_LANES = 16
_CG = 2


def _pair_ids(rng, y, x, H, W):
    positive_ids = []
    negative_ids = []
    ix_nw = 0
    iy_nw = 0
    ix_se = ix_nw + 1
    iy_se = iy_nw + 1
    for _x in range(ix_nw, ix_se + 1):
        for _y in range(iy_nw, iy_se + 1):
            if 0 <= _x <= W and 0 <= _y <= H:
                f = (_x // _RF, _y // _RF)
                if f not in positive_ids:
                    positive_ids.append((_x, _y))
    iys = rng.choices(list(range(0, H // _RF)), k=10)
    ixs = rng.choices(list(range(0, W // _RF)), k=10)
    for cx, cy in zip(ixs, iys):
        if (cx, cy) in positive_ids:
            continue
        negative_ids.append((cx, cy))
    if len(positive_ids) > _N_POSITIVE:
        positive_ids = sorted(
            positive_ids, key=lambda e: (e[1] - y) ** 2 + (e[0] - x) ** 2
        )[:_N_POSITIVE]
    if len(negative_ids) > _N_POSITIVE * _K:
        negative_ids = list(
            sorted(negative_ids, key=lambda e: (e[1] - y) ** 2 + (e[0] - x) ** 2)
        )[::-1][: _N_POSITIVE * _K]
    return positive_ids, negative_ids


@functools.lru_cache(maxsize=None)
def _build_tables(B, H, W, n_workers):
    rng = _pyrandom.Random(0)
    Hf, Wf = H // _RF, W // _RF
    ncell = Hf * Wf
    max_n = _N_POSITIVE * _K
    nloc = np.zeros((B, 2 * ncell), np.int32)
    m0 = np.zeros((B, ncell), np.float32)
    m1 = np.zeros((B, ncell), np.float32)
    M = 0
    for b in range(B):
        for h in range(Hf):
            for w in range(Wf):
                p_ids, n_ids = _pair_ids(rng, h * _RF, w * _RF, H, W)
                if len(p_ids) == 0 or len(n_ids) == 0:
                    continue
                M += 1
                i = h * Wf + w
                ny = [e[1] for e in n_ids]
                nx = [e[0] for e in n_ids]
                m = [1.0] * len(n_ids)
                while len(ny) < max_n:
                    ny.append(0)
                    nx.append(0)
                    m.append(0.0)
                nloc[b, i] = ny[0] * Wf + nx[0]
                nloc[b, ncell + i] = ny[1] * Wf + nx[1]
                m0[b, i] = m[0]
                m1[b, i] = m[1]
    cnt = np.maximum(m0 + m1, 1.0)
    scale = 1.0 / (cnt * (1e-6 + M))
    w0 = (m0 * scale).reshape(-1)
    w1 = (m1 * scale).reshape(-1)
    per_w2 = (B * ncell) // n_workers
    w_slab = np.stack(
        [w0.reshape(n_workers, per_w2), w1.reshape(n_workers, per_w2)], axis=1
    ).copy()
    return nloc, w_slab, M


def _phase0_kernel(n_workers, n_cores, B, H, W, Hf, Wf):
    ncell = Hf * Wf
    wpb = n_workers // B
    rows_per_w = Hf // wpb
    cc_per_row = Wf // _LANES

    def body(g_hbm, out_hbm, grow_v, pl_v, sem):
        wid = lax.axis_index("s") * n_cores + lax.axis_index("c")
        b = wid // wpb
        h0 = (wid % wpb) * rows_per_w
        for j in range(rows_per_w):
            pltpu.async_copy(
                g_hbm.at[b * H + (h0 + j) * _RF], grow_v.at[j], sem
            )
        for j in range(rows_per_w):
            pltpu.make_async_copy(g_hbm.at[0], grow_v.at[j], sem).wait()
        iot = lax.iota(jnp.int32, _LANES)
        for j in range(rows_per_w):
            jvec = jnp.full((_LANES,), j, jnp.int32)
            for cc in range(cc_per_row):
                base_w = (iot + cc * _LANES) * (2 * _RF)
                gx = plsc.load_gather(grow_v, [jvec, base_w])
                gy = plsc.load_gather(grow_v, [jvec, base_w + 1])
                pidx = gy.astype(jnp.int32) * Wf + gx.astype(jnp.int32)
                pidx = jnp.minimum(jnp.maximum(pidx, 0), ncell - 1)
                pl_v[pl.ds((j * cc_per_row + cc) * _LANES, _LANES)] = pidx
        seg = rows_per_w * Wf
        pltpu.sync_copy(pl_v, out_hbm.at[pl.ds(wid * seg, seg)])

    return pl.kernel(
        body,
        out_type=jax.ShapeDtypeStruct((B * ncell,), jnp.int32),
        mesh=plsc.VectorSubcoreMesh(core_axis_name="c", subcore_axis_name="s"),
        compiler_params=pltpu.CompilerParams(needs_layout_passes=False),
        scratch_types=[
            pltpu.VMEM((rows_per_w, W * 2), jnp.float32),
            pltpu.VMEM((rows_per_w * Wf,), jnp.int32),
            pltpu.SemaphoreType.DMA,
        ],
    )


def _phase1_kernel(n_workers, n_cores, B, C, ncell, Wf):
    wpb = n_workers // B
    cw_ = C // wpb
    n_stages = cw_ // _CG
    n_cc = ncell // _LANES

    def body(sk_hbm, rf_hbm, ploc_hbm, nl_hbm, out_hbm,
             nl_v, ploc_v,
             r0_v, s0_v, r1_v, s1_v,
             adp_v, an0_v, an1_v, sem0, sem1):
        wid = lax.axis_index("s") * n_cores + lax.axis_index("c")
        b = wid // wpb
        ch0 = (wid % wpb) * cw_
        pltpu.sync_copy(ploc_hbm.at[pl.ds(b * ncell, ncell)], ploc_v)
        pltpu.sync_copy(nl_hbm.at[b], nl_v)

        zz = jnp.zeros((_LANES,), jnp.float32)

        def init_loop(cc, carry):
            base = pl.multiple_of(cc * _LANES, _LANES)
            adp_v[pl.ds(base, _LANES)] = zz
            an0_v[pl.ds(base, _LANES)] = zz
            an1_v[pl.ds(base, _LANES)] = zz
            return carry

        lax.fori_loop(0, n_cc, init_loop, 0)

        sets = ((r0_v, s0_v, sem0), (r1_v, s1_v, sem1))

        def issue(s, st):
            r_v, s_v, sem = st
            c0 = ch0 + s * _CG
            pltpu.async_copy(rf_hbm.at[b, pl.ds(c0, _CG)], r_v, sem)
            pltpu.async_copy(sk_hbm.at[b, pl.ds(c0, _CG)], s_v, sem)

        def drain(st):
            r_v, s_v, sem = st
            pltpu.make_async_copy(rf_hbm.at[0, pl.ds(0, _CG)], r_v, sem).wait()
            pltpu.make_async_copy(sk_hbm.at[0, pl.ds(0, _CG)], s_v, sem).wait()

        def compute(st):
            r_v, s_v, sem = st

            def cc_body(cc, carry):
                base = pl.multiple_of(cc * _LANES, _LANES)
                hvec = jnp.full((_LANES,), base // Wf, jnp.int32)
                wvec = lax.iota(jnp.int32, _LANES) + (base % Wf)
                pvec = ploc_v[pl.ds(base, _LANES)]
                phv, pwv = pvec // Wf, pvec % Wf
                n0vec = nl_v[pl.ds(base, _LANES)]
                n0h, n0w = n0vec // Wf, n0vec % Wf
                n1vec = nl_v[pl.ds(ncell + base, _LANES)]
                n1h, n1w = n1vec // Wf, n1vec % Wf
                dp = adp_v[pl.ds(base, _LANES)]
                dn0 = an0_v[pl.ds(base, _LANES)]
                dn1 = an1_v[pl.ds(base, _LANES)]
                for k in range(_CG):
                    kvec = jnp.full((_LANES,), k, jnp.int32)
                    rv = plsc.load_gather(r_v, [kvec, hvec, wvec])
                    sp = plsc.load_gather(s_v, [kvec, phv, pwv])
                    s0 = plsc.load_gather(s_v, [kvec, n0h, n0w])
                    s1 = plsc.load_gather(s_v, [kvec, n1h, n1w])
                    d = rv - sp
                    dp = dp + d * d
                    d = rv - s0
                    dn0 = dn0 + d * d
                    d = rv - s1
                    dn1 = dn1 + d * d
                adp_v[pl.ds(base, _LANES)] = dp
                an0_v[pl.ds(base, _LANES)] = dn0
                an1_v[pl.ds(base, _LANES)] = dn1
                return carry

            lax.fori_loop(0, n_cc, cc_body, 0)

        issue(0, sets[0])
        for s in range(n_stages):
            st = sets[s % 2]
            if s + 1 < n_stages:
                issue(s + 1, sets[(s + 1) % 2])
            drain(st)
            compute(st)

        pltpu.sync_copy(adp_v, out_hbm.at[wid * 3 + 0])
        pltpu.sync_copy(an0_v, out_hbm.at[wid * 3 + 1])
        pltpu.sync_copy(an1_v, out_hbm.at[wid * 3 + 2])

    return pl.kernel(
        body,
        out_type=jax.ShapeDtypeStruct((n_workers * 3, ncell), jnp.float32),
        mesh=plsc.VectorSubcoreMesh(core_axis_name="c", subcore_axis_name="s"),
        compiler_params=pltpu.CompilerParams(needs_layout_passes=False),
        scratch_types=[
            pltpu.VMEM((2 * ncell,), jnp.int32),
            pltpu.VMEM((ncell,), jnp.int32),
            pltpu.VMEM((_CG, ncell // Wf, Wf), jnp.float32),
            pltpu.VMEM((_CG, ncell // Wf, Wf), jnp.float32),
            pltpu.VMEM((_CG, ncell // Wf, Wf), jnp.float32),
            pltpu.VMEM((_CG, ncell // Wf, Wf), jnp.float32),
            pltpu.VMEM((ncell,), jnp.float32),
            pltpu.VMEM((ncell,), jnp.float32),
            pltpu.VMEM((ncell,), jnp.float32),
            pltpu.SemaphoreType.DMA,
            pltpu.SemaphoreType.DMA,
        ],
    )


def _phase2_kernel(n_workers, n_cores, B, ncell):
    wpb = n_workers // B
    per_w = (B * ncell) // n_workers
    n_cc = per_w // _LANES

    def body(part_hbm, w_hbm, out_hbm, buf_v, w_v, out_v, sem):
        wid = lax.axis_index("s") * n_cores + lax.axis_index("c")
        b = (wid * per_w) // ncell
        lbase = wid * per_w - b * ncell
        pltpu.sync_copy(w_hbm.at[wid], w_v)
        for k in range(wpb):
            pltpu.async_copy(
                part_hbm.at[b * wpb + k, :, pl.ds(lbase, per_w)],
                buf_v.at[k],
                sem,
            )
        for k in range(wpb):
            pltpu.make_async_copy(
                part_hbm.at[0, :, pl.ds(0, per_w)], buf_v.at[k], sem
            ).wait()

        def cc_body(cc, tot):
            base = pl.multiple_of(cc * _LANES, _LANES)
            z = jnp.zeros((_LANES,), jnp.float32)
            dp, dn0, dn1 = z, z, z
            for k in range(wpb):
                dp = dp + buf_v[k, 0, pl.ds(base, _LANES)]
                dn0 = dn0 + buf_v[k, 1, pl.ds(base, _LANES)]
                dn1 = dn1 + buf_v[k, 2, pl.ds(base, _LANES)]
            w0 = w_v[0, pl.ds(base, _LANES)]
            w1 = w_v[1, pl.ds(base, _LANES)]
            return tot + (
                jnp.maximum(dp - dn0 + _MARGIN, 0.0) * w0
                + jnp.maximum(dp - dn1 + _MARGIN, 0.0) * w1
            )

        tot = lax.fori_loop(0, n_cc, cc_body, jnp.zeros((_LANES,), jnp.float32))
        out_v[...] = tot
        pltpu.sync_copy(out_v, out_hbm.at[wid])

    return pl.kernel(
        body,
        out_type=jax.ShapeDtypeStruct((n_workers, _LANES), jnp.float32),
        mesh=plsc.VectorSubcoreMesh(core_axis_name="c", subcore_axis_name="s"),
        compiler_params=pltpu.CompilerParams(needs_layout_passes=False),
        scratch_types=[
            pltpu.VMEM((wpb, 3, per_w), jnp.float32),
            pltpu.VMEM((2, per_w), jnp.float32),
            pltpu.VMEM((_LANES,), jnp.float32),
            pltpu.SemaphoreType.DMA,
        ],
    )


def kernel(sketch_context_vectors, ref_context_vectors, G):
    B, H, W, _ = G.shape
    _, C, Hf, Wf = sketch_context_vectors.shape
    ncell = Hf * Wf
    info = plsc.get_sparse_core_info()
    n_cores, n_subcores = info.num_cores, info.num_subcores
    n_workers = n_cores * n_subcores

    nloc, w_slab, M = _build_tables(int(B), int(H), int(W), n_workers)

    p0 = _phase0_kernel(
        n_workers, n_cores, int(B), int(H), int(W), int(Hf), int(Wf)
    )
    ploc = p0(jnp.reshape(G, (B * H, W * 2)))
    p1 = _phase1_kernel(n_workers, n_cores, int(B), int(C), int(ncell), int(Wf))
    partial = p1(
        sketch_context_vectors, ref_context_vectors, ploc, jnp.asarray(nloc)
    )
    partial = partial.reshape(n_workers, 3, ncell)
    p2 = _phase2_kernel(n_workers, n_cores, int(B), int(ncell))
    out = p2(partial, jnp.asarray(w_slab))
    return jnp.sum(out)


# speedup vs baseline: 1.2479x; 1.2479x over previous
import functools
import random as _pyrandom

import numpy as np
import jax
import jax.numpy as jnp
from jax import lax
from jax.experimental import pallas as pl
from jax.experimental.pallas import tpu as pltpu
from jax.experimental.pallas import tpu_sc as plsc

_RF = 8
_N_POSITIVE = 2
_K = 1
_MARGIN = 12.0
_LANES = 16
_CG = 2


def _pair_ids(rng, y, x, H, W):
    positive_ids = []
    negative_ids = []
    ix_nw = 0
    iy_nw = 0
    ix_se = ix_nw + 1
    iy_se = iy_nw + 1
    for _x in range(ix_nw, ix_se + 1):
        for _y in range(iy_nw, iy_se + 1):
            if 0 <= _x <= W and 0 <= _y <= H:
                f = (_x // _RF, _y // _RF)
                if f not in positive_ids:
                    positive_ids.append((_x, _y))
    iys = rng.choices(list(range(0, H // _RF)), k=10)
    ixs = rng.choices(list(range(0, W // _RF)), k=10)
    for cx, cy in zip(ixs, iys):
        if (cx, cy) in positive_ids:
            continue
        negative_ids.append((cx, cy))
    if len(positive_ids) > _N_POSITIVE:
        positive_ids = sorted(
            positive_ids, key=lambda e: (e[1] - y) ** 2 + (e[0] - x) ** 2
        )[:_N_POSITIVE]
    if len(negative_ids) > _N_POSITIVE * _K:
        negative_ids = list(
            sorted(negative_ids, key=lambda e: (e[1] - y) ** 2 + (e[0] - x) ** 2)
        )[::-1][: _N_POSITIVE * _K]
    return positive_ids, negative_ids


@functools.lru_cache(maxsize=None)
def _build_tables(B, H, W, n_workers):
    rng = _pyrandom.Random(0)
    Hf, Wf = H // _RF, W // _RF
    ncell = Hf * Wf
    max_n = _N_POSITIVE * _K
    nh = np.zeros((B, 2 * ncell), np.int32)
    nw = np.zeros((B, 2 * ncell), np.int32)
    m0 = np.zeros((B, ncell), np.float32)
    m1 = np.zeros((B, ncell), np.float32)
    M = 0
    for b in range(B):
        for h in range(Hf):
            for w in range(Wf):
                p_ids, n_ids = _pair_ids(rng, h * _RF, w * _RF, H, W)
                if len(p_ids) == 0 or len(n_ids) == 0:
                    continue
                M += 1
                i = h * Wf + w
                ny = [e[1] for e in n_ids]
                nx = [e[0] for e in n_ids]
                m = [1.0] * len(n_ids)
                while len(ny) < max_n:
                    ny.append(0)
                    nx.append(0)
                    m.append(0.0)
                nh[b, i] = ny[0]
                nw[b, i] = nx[0]
                nh[b, ncell + i] = ny[1]
                nw[b, ncell + i] = nx[1]
                m0[b, i] = m[0]
                m1[b, i] = m[1]
    cnt = np.maximum(m0 + m1, 1.0)
    scale = 1.0 / (cnt * (1e-6 + M))
    w0 = (m0 * scale).reshape(-1)
    w1 = (m1 * scale).reshape(-1)
    per_w2 = (B * ncell) // n_workers
    w_slab = np.stack(
        [w0.reshape(n_workers, per_w2), w1.reshape(n_workers, per_w2)], axis=1
    ).copy()
    return nh, nw, w_slab, M


def _phase0_kernel(n_workers, n_cores, B, H, W, Hf, Wf):
    ncell = Hf * Wf
    wpb = n_workers // B
    rows_per_w = Hf // wpb
    cc_per_row = Wf // _LANES

    def body(g_hbm, out_hbm, grow_v, py_v, px_v, sem):
        wid = lax.axis_index("s") * n_cores + lax.axis_index("c")
        b = wid // wpb
        h0 = (wid % wpb) * rows_per_w
        for j in range(rows_per_w):
            pltpu.async_copy(
                g_hbm.at[b * H + (h0 + j) * _RF], grow_v.at[j], sem
            )
        for j in range(rows_per_w):
            pltpu.make_async_copy(g_hbm.at[0], grow_v.at[j], sem).wait()
        iot = lax.iota(jnp.int32, _LANES)
        for j in range(rows_per_w):
            jvec = jnp.full((_LANES,), j, jnp.int32)
            for cc in range(cc_per_row):
                base_w = (iot + cc * _LANES) * (2 * _RF)
                gx = plsc.load_gather(grow_v, [jvec, base_w])
                gy = plsc.load_gather(grow_v, [jvec, base_w + 1])
                py = jnp.minimum(jnp.maximum(gy.astype(jnp.int32), 0), Hf - 1)
                px = jnp.minimum(jnp.maximum(gx.astype(jnp.int32), 0), Wf - 1)
                off = (j * cc_per_row + cc) * _LANES
                py_v[pl.ds(off, _LANES)] = py
                px_v[pl.ds(off, _LANES)] = px
        seg = rows_per_w * Wf
        pltpu.sync_copy(py_v, out_hbm.at[pl.ds(wid * seg, seg)])
        pltpu.sync_copy(px_v, out_hbm.at[pl.ds(B * ncell + wid * seg, seg)])

    return pl.kernel(
        body,
        out_type=jax.ShapeDtypeStruct((2 * B * ncell,), jnp.int32),
        mesh=plsc.VectorSubcoreMesh(core_axis_name="c", subcore_axis_name="s"),
        compiler_params=pltpu.CompilerParams(needs_layout_passes=False),
        scratch_types=[
            pltpu.VMEM((rows_per_w, W * 2), jnp.float32),
            pltpu.VMEM((rows_per_w * Wf,), jnp.int32),
            pltpu.VMEM((rows_per_w * Wf,), jnp.int32),
            pltpu.SemaphoreType.DMA,
        ],
    )


def _phase1_kernel(n_workers, n_cores, B, C, ncell, Wf):
    wpb = n_workers // B
    cw_ = C // wpb
    n_stages = cw_ // _CG
    n_cc = ncell // _LANES

    def body(sk_hbm, rf_hbm, ploc_hbm, nh_hbm, nw_hbm, out_hbm,
             ph_v, pw_v, nh_v, nw_v,
             r0_v, s0_v, r1_v, s1_v,
             adp_v, an0_v, an1_v, sem0, sem1):
        wid = lax.axis_index("s") * n_cores + lax.axis_index("c")
        b = wid // wpb
        ch0 = (wid % wpb) * cw_
        nb = B * ncell
        pltpu.sync_copy(ploc_hbm.at[pl.ds(b * ncell, ncell)], ph_v)
        pltpu.sync_copy(ploc_hbm.at[pl.ds(nb + b * ncell, ncell)], pw_v)
        pltpu.sync_copy(nh_hbm.at[b], nh_v)
        pltpu.sync_copy(nw_hbm.at[b], nw_v)

        zz = jnp.zeros((_LANES,), jnp.float32)

        def init_loop(cc, carry):
            base = pl.multiple_of(cc * _LANES, _LANES)
            adp_v[pl.ds(base, _LANES)] = zz
            an0_v[pl.ds(base, _LANES)] = zz
            an1_v[pl.ds(base, _LANES)] = zz
            return carry

        lax.fori_loop(0, n_cc, init_loop, 0)

        sets = ((r0_v, s0_v, sem0), (r1_v, s1_v, sem1))

        def issue(s, st):
            r_v, s_v, sem = st
            c0 = ch0 + s * _CG
            pltpu.async_copy(rf_hbm.at[b, pl.ds(c0, _CG)], r_v, sem)
            pltpu.async_copy(sk_hbm.at[b, pl.ds(c0, _CG)], s_v, sem)

        def drain(st):
            r_v, s_v, sem = st
            pltpu.make_async_copy(rf_hbm.at[0, pl.ds(0, _CG)], r_v, sem).wait()
            pltpu.make_async_copy(sk_hbm.at[0, pl.ds(0, _CG)], s_v, sem).wait()

        def compute(st):
            r_v, s_v, sem = st

            def cc_body(cc, carry):
                base = pl.multiple_of(cc * _LANES, _LANES)
                hvec = jnp.full((_LANES,), base // Wf, jnp.int32)
                wvec = lax.iota(jnp.int32, _LANES) + (base % Wf)
                phv = ph_v[pl.ds(base, _LANES)]
                pwv = pw_v[pl.ds(base, _LANES)]
                n0h = nh_v[pl.ds(base, _LANES)]
                n0w = nw_v[pl.ds(base, _LANES)]
                n1h = nh_v[pl.ds(ncell + base, _LANES)]
                n1w = nw_v[pl.ds(ncell + base, _LANES)]
                dp = adp_v[pl.ds(base, _LANES)]
                dn0 = an0_v[pl.ds(base, _LANES)]
                dn1 = an1_v[pl.ds(base, _LANES)]
                for k in range(_CG):
                    kvec = jnp.full((_LANES,), k, jnp.int32)
                    rv = plsc.load_gather(r_v, [kvec, hvec, wvec])
                    sp = plsc.load_gather(s_v, [kvec, phv, pwv])
                    s0 = plsc.load_gather(s_v, [kvec, n0h, n0w])
                    s1 = plsc.load_gather(s_v, [kvec, n1h, n1w])
                    d = rv - sp
                    dp = dp + d * d
                    d = rv - s0
                    dn0 = dn0 + d * d
                    d = rv - s1
                    dn1 = dn1 + d * d
                adp_v[pl.ds(base, _LANES)] = dp
                an0_v[pl.ds(base, _LANES)] = dn0
                an1_v[pl.ds(base, _LANES)] = dn1
                return carry

            lax.fori_loop(0, n_cc, cc_body, 0)

        issue(0, sets[0])
        for s in range(n_stages):
            st = sets[s % 2]
            if s + 1 < n_stages:
                issue(s + 1, sets[(s + 1) % 2])
            drain(st)
            compute(st)

        pltpu.sync_copy(adp_v, out_hbm.at[wid * 3 + 0])
        pltpu.sync_copy(an0_v, out_hbm.at[wid * 3 + 1])
        pltpu.sync_copy(an1_v, out_hbm.at[wid * 3 + 2])

    return pl.kernel(
        body,
        out_type=jax.ShapeDtypeStruct((n_workers * 3, ncell), jnp.float32),
        mesh=plsc.VectorSubcoreMesh(core_axis_name="c", subcore_axis_name="s"),
        compiler_params=pltpu.CompilerParams(needs_layout_passes=False),
        scratch_types=[
            pltpu.VMEM((ncell,), jnp.int32),
            pltpu.VMEM((ncell,), jnp.int32),
            pltpu.VMEM((2 * ncell,), jnp.int32),
            pltpu.VMEM((2 * ncell,), jnp.int32),
            pltpu.VMEM((_CG, ncell // Wf, Wf), jnp.float32),
            pltpu.VMEM((_CG, ncell // Wf, Wf), jnp.float32),
            pltpu.VMEM((_CG, ncell // Wf, Wf), jnp.float32),
            pltpu.VMEM((_CG, ncell // Wf, Wf), jnp.float32),
            pltpu.VMEM((ncell,), jnp.float32),
            pltpu.VMEM((ncell,), jnp.float32),
            pltpu.VMEM((ncell,), jnp.float32),
            pltpu.SemaphoreType.DMA,
            pltpu.SemaphoreType.DMA,
        ],
    )


def _phase2_kernel(n_workers, n_cores, B, ncell):
    wpb = n_workers // B
    per_w = (B * ncell) // n_workers
    n_cc = per_w // _LANES

    def body(part_hbm, w_hbm, out_hbm, buf_v, w_v, out_v, sem):
        wid = lax.axis_index("s") * n_cores + lax.axis_index("c")
        b = (wid * per_w) // ncell
        lbase = wid * per_w - b * ncell
        pltpu.sync_copy(w_hbm.at[wid], w_v)
        for k in range(wpb):
            pltpu.async_copy(
                part_hbm.at[b * wpb + k, :, pl.ds(lbase, per_w)],
                buf_v.at[k],
                sem,
            )
        for k in range(wpb):
            pltpu.make_async_copy(
                part_hbm.at[0, :, pl.ds(0, per_w)], buf_v.at[k], sem
            ).wait()

        def cc_body(cc, tot):
            base = pl.multiple_of(cc * _LANES, _LANES)
            z = jnp.zeros((_LANES,), jnp.float32)
            dp, dn0, dn1 = z, z, z
            for k in range(wpb):
                dp = dp + buf_v[k, 0, pl.ds(base, _LANES)]
                dn0 = dn0 + buf_v[k, 1, pl.ds(base, _LANES)]
                dn1 = dn1 + buf_v[k, 2, pl.ds(base, _LANES)]
            w0 = w_v[0, pl.ds(base, _LANES)]
            w1 = w_v[1, pl.ds(base, _LANES)]
            return tot + (
                jnp.maximum(dp - dn0 + _MARGIN, 0.0) * w0
                + jnp.maximum(dp - dn1 + _MARGIN, 0.0) * w1
            )

        tot = lax.fori_loop(0, n_cc, cc_body, jnp.zeros((_LANES,), jnp.float32))
        out_v[...] = tot
        pltpu.sync_copy(out_v, out_hbm.at[wid])

    return pl.kernel(
        body,
        out_type=jax.ShapeDtypeStruct((n_workers, _LANES), jnp.float32),
        mesh=plsc.VectorSubcoreMesh(core_axis_name="c", subcore_axis_name="s"),
        compiler_params=pltpu.CompilerParams(needs_layout_passes=False),
        scratch_types=[
            pltpu.VMEM((wpb, 3, per_w), jnp.float32),
            pltpu.VMEM((2, per_w), jnp.float32),
            pltpu.VMEM((_LANES,), jnp.float32),
            pltpu.SemaphoreType.DMA,
        ],
    )


def kernel(sketch_context_vectors, ref_context_vectors, G):
    B, H, W, _ = G.shape
    _, C, Hf, Wf = sketch_context_vectors.shape
    ncell = Hf * Wf
    info = plsc.get_sparse_core_info()
    n_cores, n_subcores = info.num_cores, info.num_subcores
    n_workers = n_cores * n_subcores

    nh, nw, w_slab, M = _build_tables(int(B), int(H), int(W), n_workers)

    p0 = _phase0_kernel(
        n_workers, n_cores, int(B), int(H), int(W), int(Hf), int(Wf)
    )
    ploc = p0(jnp.reshape(G, (B * H, W * 2)))
    p1 = _phase1_kernel(n_workers, n_cores, int(B), int(C), int(ncell), int(Wf))
    partial = p1(
        sketch_context_vectors, ref_context_vectors, ploc,
        jnp.asarray(nh), jnp.asarray(nw),
    )
    partial = partial.reshape(n_workers, 3, ncell)
    p2 = _phase2_kernel(n_workers, n_cores, int(B), int(ncell))
    out = p2(partial, jnp.asarray(w_slab))
    return jnp.sum(out)


# speedup vs baseline: 1.6681x; 1.3368x over previous
import functools
import random as _pyrandom

import numpy as np
import jax
import jax.numpy as jnp
from jax import lax
from jax.experimental import pallas as pl
from jax.experimental.pallas import tpu as pltpu
from jax.experimental.pallas import tpu_sc as plsc

_RF = 8
_N_POSITIVE = 2
_K = 1
_MARGIN = 12.0
_LANES = 16
_CG = 4


def _pair_ids(rng, y, x, H, W):
    positive_ids = []
    negative_ids = []
    ix_nw = 0
    iy_nw = 0
    ix_se = ix_nw + 1
    iy_se = iy_nw + 1
    for _x in range(ix_nw, ix_se + 1):
        for _y in range(iy_nw, iy_se + 1):
            if 0 <= _x <= W and 0 <= _y <= H:
                f = (_x // _RF, _y // _RF)
                if f not in positive_ids:
                    positive_ids.append((_x, _y))
    iys = rng.choices(list(range(0, H // _RF)), k=10)
    ixs = rng.choices(list(range(0, W // _RF)), k=10)
    for cx, cy in zip(ixs, iys):
        if (cx, cy) in positive_ids:
            continue
        negative_ids.append((cx, cy))
    if len(positive_ids) > _N_POSITIVE:
        positive_ids = sorted(
            positive_ids, key=lambda e: (e[1] - y) ** 2 + (e[0] - x) ** 2
        )[:_N_POSITIVE]
    if len(negative_ids) > _N_POSITIVE * _K:
        negative_ids = list(
            sorted(negative_ids, key=lambda e: (e[1] - y) ** 2 + (e[0] - x) ** 2)
        )[::-1][: _N_POSITIVE * _K]
    return positive_ids, negative_ids


@functools.lru_cache(maxsize=None)
def _build_tables(B, H, W, n_workers):
    rng = _pyrandom.Random(0)
    Hf, Wf = H // _RF, W // _RF
    ncell = Hf * Wf
    max_n = _N_POSITIVE * _K
    nloc = np.zeros((B, 2 * ncell), np.int32)
    m0 = np.zeros((B, ncell), np.float32)
    m1 = np.zeros((B, ncell), np.float32)
    M = 0
    for b in range(B):
        for h in range(Hf):
            for w in range(Wf):
                p_ids, n_ids = _pair_ids(rng, h * _RF, w * _RF, H, W)
                if len(p_ids) == 0 or len(n_ids) == 0:
                    continue
                M += 1
                i = h * Wf + w
                ny = [e[1] for e in n_ids]
                nx = [e[0] for e in n_ids]
                m = [1.0] * len(n_ids)
                while len(ny) < max_n:
                    ny.append(0)
                    nx.append(0)
                    m.append(0.0)
                nloc[b, i] = ny[0] * Wf + nx[0]
                nloc[b, ncell + i] = ny[1] * Wf + nx[1]
                m0[b, i] = m[0]
                m1[b, i] = m[1]
    cnt = np.maximum(m0 + m1, 1.0)
    scale = 1.0 / (cnt * (1e-6 + M))
    w0 = (m0 * scale).reshape(-1)
    w1 = (m1 * scale).reshape(-1)
    per_w2 = (B * ncell) // n_workers
    w_slab = np.stack(
        [w0.reshape(n_workers, per_w2), w1.reshape(n_workers, per_w2)], axis=1
    ).copy()
    return nloc, w_slab, M


def _phase0_kernel(n_workers, n_cores, B, H, W, Hf, Wf):
    ncell = Hf * Wf
    wpb = n_workers // B
    rows_per_w = Hf // wpb
    cc_per_row = Wf // _LANES

    def body(g_hbm, out_hbm, grow_v, pl_v, sem):
        wid = lax.axis_index("s") * n_cores + lax.axis_index("c")
        b = wid // wpb
        h0 = (wid % wpb) * rows_per_w
        for j in range(rows_per_w):
            pltpu.async_copy(
                g_hbm.at[b * H + (h0 + j) * _RF], grow_v.at[j], sem
            )
        for j in range(rows_per_w):
            pltpu.make_async_copy(g_hbm.at[0], grow_v.at[j], sem).wait()
        iot = lax.iota(jnp.int32, _LANES)
        for j in range(rows_per_w):
            jvec = jnp.full((_LANES,), j, jnp.int32)
            for cc in range(cc_per_row):
                base_w = (iot + cc * _LANES) * (2 * _RF)
                gx = plsc.load_gather(grow_v, [jvec, base_w])
                gy = plsc.load_gather(grow_v, [jvec, base_w + 1])
                pidx = gy.astype(jnp.int32) * Wf + gx.astype(jnp.int32)
                pidx = jnp.minimum(jnp.maximum(pidx, 0), ncell - 1)
                pl_v[pl.ds((j * cc_per_row + cc) * _LANES, _LANES)] = pidx
        seg = rows_per_w * Wf
        pltpu.sync_copy(pl_v, out_hbm.at[pl.ds(wid * seg, seg)])

    return pl.kernel(
        body,
        out_type=jax.ShapeDtypeStruct((B * ncell,), jnp.int32),
        mesh=plsc.VectorSubcoreMesh(core_axis_name="c", subcore_axis_name="s"),
        compiler_params=pltpu.CompilerParams(needs_layout_passes=False),
        scratch_types=[
            pltpu.VMEM((rows_per_w, W * 2), jnp.float32),
            pltpu.VMEM((rows_per_w * Wf,), jnp.int32),
            pltpu.SemaphoreType.DMA,
        ],
    )


def _phase1_kernel(n_workers, n_cores, B, C, ncell, Wf):
    wpb = n_workers // B
    cw_ = C // wpb
    n_stages = cw_ // _CG
    n_cc = ncell // _LANES

    def body(sk_hbm, rf_hbm, ploc_hbm, nl_hbm, out_hbm,
             ploc_v, nl_v,
             r0_v, s0_v, r1_v, s1_v,
             adp_v, an0_v, an1_v, sem0, sem1):
        wid = lax.axis_index("s") * n_cores + lax.axis_index("c")
        b = wid // wpb
        ch0 = (wid % wpb) * cw_
        pltpu.sync_copy(ploc_hbm.at[pl.ds(b * ncell, ncell)], ploc_v)
        pltpu.sync_copy(nl_hbm.at[b], nl_v)

        zz = jnp.zeros((_LANES,), jnp.float32)

        def init_loop(cc, carry):
            base = pl.multiple_of(cc * _LANES, _LANES)
            adp_v[pl.ds(base, _LANES)] = zz
            an0_v[pl.ds(base, _LANES)] = zz
            an1_v[pl.ds(base, _LANES)] = zz
            return carry

        lax.fori_loop(0, n_cc, init_loop, 0)

        sets = ((r0_v, s0_v, sem0), (r1_v, s1_v, sem1))

        def issue(s, st):
            r_v, s_v, sem = st
            c0 = ch0 + s * _CG
            pltpu.async_copy(rf_hbm.at[b, pl.ds(c0, _CG)], r_v, sem)
            pltpu.async_copy(sk_hbm.at[b, pl.ds(c0, _CG)], s_v, sem)

        def drain(st):
            r_v, s_v, sem = st
            pltpu.make_async_copy(rf_hbm.at[0, pl.ds(0, _CG)], r_v, sem).wait()
            pltpu.make_async_copy(sk_hbm.at[0, pl.ds(0, _CG)], s_v, sem).wait()

        def compute(st):
            r_v, s_v, sem = st

            def cc_body(cc, carry):
                base = pl.multiple_of(cc * _LANES, _LANES)
                pvec = ploc_v[pl.ds(base, _LANES)]
                n0vec = nl_v[pl.ds(base, _LANES)]
                n1vec = nl_v[pl.ds(ncell + base, _LANES)]
                dp = adp_v[pl.ds(base, _LANES)]
                dn0 = an0_v[pl.ds(base, _LANES)]
                dn1 = an1_v[pl.ds(base, _LANES)]
                for k in range(_CG):
                    kvec = jnp.full((_LANES,), k, jnp.int32)
                    rv = r_v[k, pl.ds(base, _LANES)]
                    sp = plsc.load_gather(s_v, [kvec, pvec])
                    s0 = plsc.load_gather(s_v, [kvec, n0vec])
                    s1 = plsc.load_gather(s_v, [kvec, n1vec])
                    d = rv - sp
                    dp = dp + d * d
                    d = rv - s0
                    dn0 = dn0 + d * d
                    d = rv - s1
                    dn1 = dn1 + d * d
                adp_v[pl.ds(base, _LANES)] = dp
                an0_v[pl.ds(base, _LANES)] = dn0
                an1_v[pl.ds(base, _LANES)] = dn1
                return carry

            lax.fori_loop(0, n_cc, cc_body, 0)

        issue(0, sets[0])
        for s in range(n_stages):
            st = sets[s % 2]
            if s + 1 < n_stages:
                issue(s + 1, sets[(s + 1) % 2])
            drain(st)
            compute(st)

        pltpu.sync_copy(adp_v, out_hbm.at[wid * 3 + 0])
        pltpu.sync_copy(an0_v, out_hbm.at[wid * 3 + 1])
        pltpu.sync_copy(an1_v, out_hbm.at[wid * 3 + 2])

    return pl.kernel(
        body,
        out_type=jax.ShapeDtypeStruct((n_workers * 3, ncell), jnp.float32),
        mesh=plsc.VectorSubcoreMesh(core_axis_name="c", subcore_axis_name="s"),
        compiler_params=pltpu.CompilerParams(needs_layout_passes=False),
        scratch_types=[
            pltpu.VMEM((ncell,), jnp.int32),
            pltpu.VMEM((2 * ncell,), jnp.int32),
            pltpu.VMEM((_CG, ncell), jnp.float32),
            pltpu.VMEM((_CG, ncell), jnp.float32),
            pltpu.VMEM((_CG, ncell), jnp.float32),
            pltpu.VMEM((_CG, ncell), jnp.float32),
            pltpu.VMEM((ncell,), jnp.float32),
            pltpu.VMEM((ncell,), jnp.float32),
            pltpu.VMEM((ncell,), jnp.float32),
            pltpu.SemaphoreType.DMA,
            pltpu.SemaphoreType.DMA,
        ],
    )


def _phase2_kernel(n_workers, n_cores, B, ncell):
    wpb = n_workers // B
    per_w = (B * ncell) // n_workers
    n_cc = per_w // _LANES

    def body(part_hbm, w_hbm, out_hbm, buf_v, w_v, out_v, sem):
        wid = lax.axis_index("s") * n_cores + lax.axis_index("c")
        b = (wid * per_w) // ncell
        lbase = wid * per_w - b * ncell
        pltpu.sync_copy(w_hbm.at[wid], w_v)
        for k in range(wpb):
            pltpu.async_copy(
                part_hbm.at[b * wpb + k, :, pl.ds(lbase, per_w)],
                buf_v.at[k],
                sem,
            )
        for k in range(wpb):
            pltpu.make_async_copy(
                part_hbm.at[0, :, pl.ds(0, per_w)], buf_v.at[k], sem
            ).wait()

        def cc_body(cc, tot):
            base = pl.multiple_of(cc * _LANES, _LANES)
            z = jnp.zeros((_LANES,), jnp.float32)
            dp, dn0, dn1 = z, z, z
            for k in range(wpb):
                dp = dp + buf_v[k, 0, pl.ds(base, _LANES)]
                dn0 = dn0 + buf_v[k, 1, pl.ds(base, _LANES)]
                dn1 = dn1 + buf_v[k, 2, pl.ds(base, _LANES)]
            w0 = w_v[0, pl.ds(base, _LANES)]
            w1 = w_v[1, pl.ds(base, _LANES)]
            return tot + (
                jnp.maximum(dp - dn0 + _MARGIN, 0.0) * w0
                + jnp.maximum(dp - dn1 + _MARGIN, 0.0) * w1
            )

        tot = lax.fori_loop(0, n_cc, cc_body, jnp.zeros((_LANES,), jnp.float32))
        out_v[...] = tot
        pltpu.sync_copy(out_v, out_hbm.at[wid])

    return pl.kernel(
        body,
        out_type=jax.ShapeDtypeStruct((n_workers, _LANES), jnp.float32),
        mesh=plsc.VectorSubcoreMesh(core_axis_name="c", subcore_axis_name="s"),
        compiler_params=pltpu.CompilerParams(needs_layout_passes=False),
        scratch_types=[
            pltpu.VMEM((wpb, 3, per_w), jnp.float32),
            pltpu.VMEM((2, per_w), jnp.float32),
            pltpu.VMEM((_LANES,), jnp.float32),
            pltpu.SemaphoreType.DMA,
        ],
    )


def kernel(sketch_context_vectors, ref_context_vectors, G):
    B, H, W, _ = G.shape
    _, C, Hf, Wf = sketch_context_vectors.shape
    ncell = Hf * Wf
    info = plsc.get_sparse_core_info()
    n_cores, n_subcores = info.num_cores, info.num_subcores
    n_workers = n_cores * n_subcores

    nloc, w_slab, M = _build_tables(int(B), int(H), int(W), n_workers)

    p0 = _phase0_kernel(
        n_workers, n_cores, int(B), int(H), int(W), int(Hf), int(Wf)
    )
    ploc = p0(jnp.reshape(G, (B * H, W * 2)))
    p1 = _phase1_kernel(n_workers, n_cores, int(B), int(C), int(ncell), int(Wf))
    sk3 = jnp.reshape(sketch_context_vectors, (B, C, ncell))
    rf3 = jnp.reshape(ref_context_vectors, (B, C, ncell))
    partial = p1(sk3, rf3, ploc, jnp.asarray(nloc))
    partial = partial.reshape(n_workers, 3, ncell)
    p2 = _phase2_kernel(n_workers, n_cores, int(B), int(ncell))
    out = p2(partial, jnp.asarray(w_slab))
    return jnp.sum(out)
